# baseline (speedup 1.0000x reference)
import functools

import jax
import jax.numpy as jnp
from jax import lax
from jax.experimental import pallas as pl
from jax.experimental.pallas import tpu as pltpu
from jax.experimental.pallas import tpu_sc as plsc

N = 10000
NPAD = 10240
E = 320000
EPAD = 327680
B = 512
D = 128
VOCAB = 100000
NC = 2
NS = 16
NW = NC * NS

_f32 = jnp.float32


def _emb_body(emb_hbm, idx_hbm, out_hbm, idx_v, rows_v, sem):
    c = lax.axis_index("c")
    s = lax.axis_index("s")
    wid = s * NC + c
    pltpu.sync_copy(idx_hbm.at[pl.ds(wid * 320, 320)], idx_v)
    for g in range(5):
        pltpu.async_copy(emb_hbm.at[idx_v.at[pl.ds(g * 64, 64)]],
                         rows_v, sem).wait()
        pltpu.sync_copy(rows_v, out_hbm.at[pl.ds(wid * 320 + g * 64, 64)])


@functools.cache
def _emb_kernel():
    return pl.kernel(
        _emb_body,
        out_type=jax.ShapeDtypeStruct((NPAD, D), _f32),
        mesh=plsc.VectorSubcoreMesh(
            core_axis_name="c", subcore_axis_name="s",
            num_cores=NC, num_subcores=NS,
        ),
        scratch_types=[
            pltpu.VMEM((320,), jnp.int32),
            pltpu.VMEM((64, D), _f32),
            pltpu.SemaphoreType.DMA,
        ],
    )


def _emb_call(emb, xi2):
    return _emb_kernel()(emb, xi2)


def _agg_body(src_hbm, dst_hbm, h_hbm, agg_hbm,
              src_v, dst_v, rows_v, agg_sh, sem, sem2, ssem, ssem2):
    c = lax.axis_index("c")
    s = lax.axis_index("s")
    wid = s * NC + c
    zero16 = jnp.zeros((16,), _f32)

    def _zr(i, carry):
        rows_v[0, i // 8, pl.ds((i % 8) * 16, 16)] = zero16
        return carry
    lax.fori_loop(0, D * D // 16, _zr, 0)
    for z in range(5):
        pltpu.sync_copy(rows_v.at[0], agg_sh.at[pl.ds(s * 640 + z * 128, 128)])
    plsc.subcore_barrier()

    def _chunk(ch, carry):
        rbase = wid * 80 + ch * 16
        pltpu.sync_copy(src_hbm.at[pl.ds(rbase, 16)], src_v)
        pltpu.sync_copy(dst_hbm.at[pl.ds(rbase, 16)], dst_v)
        pltpu.async_copy(h_hbm.at[src_v.at[0]], rows_v.at[0], sem)

        def _pair(gg, carry2):
            g0 = gg * 2
            g1 = g0 + 1
            g2 = jnp.minimum(g0 + 2, 15)
            pltpu.async_copy(h_hbm.at[src_v.at[g1]], rows_v.at[1], sem2)
            pltpu.make_async_copy(h_hbm.at[src_v.at[g0]], rows_v.at[0],
                                  sem).wait()
            pltpu.async_copy(rows_v.at[0], agg_sh.at[dst_v.at[g0]], ssem,
                             add=True)
            pltpu.make_async_copy(h_hbm.at[src_v.at[g1]], rows_v.at[1],
                                  sem2).wait()
            pltpu.async_copy(rows_v.at[1], agg_sh.at[dst_v.at[g1]], ssem2,
                             add=True)
            pltpu.make_async_copy(rows_v.at[0], agg_sh.at[dst_v.at[g0]],
                                  ssem).wait()
            pltpu.async_copy(h_hbm.at[src_v.at[g2]], rows_v.at[0], sem)
            pltpu.make_async_copy(rows_v.at[1], agg_sh.at[dst_v.at[g1]],
                                  ssem2).wait()
            return carry2
        lax.fori_loop(0, 8, _pair, 0)
        pltpu.make_async_copy(h_hbm.at[src_v.at[0]], rows_v.at[0], sem).wait()
        return carry
    lax.fori_loop(0, 5, _chunk, 0)

    plsc.subcore_barrier()
    pltpu.sync_copy(agg_sh.at[pl.ds(s * 640, 640)],
                    agg_hbm.at[c, pl.ds(s * 640, 640)])


@functools.cache
def _agg_kernel():
    return pl.kernel(
        _agg_body,
        out_type=jax.ShapeDtypeStruct((NC, NPAD, D), _f32),
        mesh=plsc.VectorSubcoreMesh(
            core_axis_name="c", subcore_axis_name="s",
            num_cores=NC, num_subcores=NS,
        ),
        scratch_types=[
            pltpu.VMEM((16, 128), jnp.int32),
            pltpu.VMEM((16, 128), jnp.int32),
            pltpu.VMEM((2, 128, D), _f32),
            pltpu.VMEM_SHARED((NPAD, D), _f32),
            pltpu.SemaphoreType.DMA,
            pltpu.SemaphoreType.DMA,
            pltpu.SemaphoreType.DMA,
            pltpu.SemaphoreType.DMA,
        ],
        compiler_params=pltpu.CompilerParams(needs_layout_passes=False),
    )


def _deg_body(src_hbm, dst_hbm, nm_hbm, deg_hbm, src_v, dst_v, nm_v, deg_v):
    c = lax.axis_index("c")
    s = lax.axis_index("s")
    wid = s * NC + c
    zero16 = jnp.zeros((16,), _f32)

    def _zd(i, carry):
        deg_v[pl.ds(i * 16, 16)] = zero16
        return carry
    lax.fori_loop(0, NPAD // 16, _zd, 0)

    pltpu.sync_copy(nm_hbm, nm_v)
    pltpu.sync_copy(src_hbm.at[pl.ds(wid * 80, 80)], src_v)
    pltpu.sync_copy(dst_hbm.at[pl.ds(wid * 80, 80)], dst_v)

    def _deg(j, carry):
        si = src_v[j // 8, pl.ds((j % 8) * 16, 16)]
        vals = plsc.load_gather(nm_v, [si])
        di = dst_v[j // 8, pl.ds((j % 8) * 16, 16)]
        plsc.addupdate_scatter(deg_v, [di], vals)
        return carry
    lax.fori_loop(0, 640, _deg, 0)
    pltpu.sync_copy(deg_v, deg_hbm.at[pl.ds(wid * NPAD, NPAD)])


@functools.cache
def _deg_kernel():
    return pl.kernel(
        _deg_body,
        out_type=jax.ShapeDtypeStruct((NW * NPAD,), _f32),
        mesh=plsc.VectorSubcoreMesh(
            core_axis_name="c", subcore_axis_name="s",
            num_cores=NC, num_subcores=NS,
        ),
        scratch_types=[
            pltpu.VMEM((80, 128), jnp.int32),
            pltpu.VMEM((80, 128), jnp.int32),
            pltpu.VMEM((NPAD,), _f32),
            pltpu.VMEM((NPAD,), _f32),
        ],
        compiler_params=pltpu.CompilerParams(needs_layout_passes=False),
    )


def _agg_call(srcp, dstp, h, nm):
    aggP = _agg_kernel()(srcp, dstp, h)
    degP = _deg_kernel()(srcp, dstp, nm)
    return aggP, degP.reshape(NW, NPAD)


def _sage_body(agg_ref, deg_ref, h_ref, nm_ref, wl_ref, bl_ref, wr_ref, p_ref,
               h1_ref, sc_ref, key_ref):
    agg = agg_ref[0] + agg_ref[1]
    deg = jnp.sum(deg_ref[...], axis=0)
    nm = nm_ref[...]
    h = h_ref[...]
    mean = agg / jnp.maximum(deg, 1.0)[:, None]
    out = (jnp.dot(mean, wl_ref[...], preferred_element_type=_f32)
           + bl_ref[...][None, :]
           + jnp.dot(h, wr_ref[...], preferred_element_type=_f32))
    h1 = jnp.maximum(out, 0.0) * nm[:, None]
    p = p_ref[...]
    pn = jax.lax.rsqrt(jnp.sum(p * p))
    sc = jnp.tanh(jnp.dot(h1, p, preferred_element_type=_f32) * pn)
    h1_ref[...] = h1
    sc_ref[...] = sc
    key_ref[...] = jnp.where(nm > 0.5, sc, -2.0)


_R = 512


def _sage_call(aggP, degP, h, nm, Wl, bl, Wr, p):
    grid = NPAD // _R
    return pl.pallas_call(
        _sage_body,
        grid=(grid,),
        in_specs=[
            pl.BlockSpec((NC, _R, D), lambda i: (0, i, 0)),
            pl.BlockSpec((NW, _R), lambda i: (0, i)),
            pl.BlockSpec((_R, D), lambda i: (i, 0)),
            pl.BlockSpec((_R,), lambda i: (i,)),
            pl.BlockSpec((D, D), lambda i: (0, 0)),
            pl.BlockSpec((D,), lambda i: (0,)),
            pl.BlockSpec((D, D), lambda i: (0, 0)),
            pl.BlockSpec((D,), lambda i: (0,)),
        ],
        out_specs=[
            pl.BlockSpec((_R, D), lambda i: (i, 0)),
            pl.BlockSpec((_R,), lambda i: (i,)),
            pl.BlockSpec((_R,), lambda i: (i,)),
        ],
        out_shape=[
            jax.ShapeDtypeStruct((NPAD, D), _f32),
            jax.ShapeDtypeStruct((NPAD,), _f32),
            jax.ShapeDtypeStruct((NPAD,), _f32),
        ],
    )(aggP, degP, h, nm, Wl, bl, Wr, p)


_NB = NPAD // 128


def _select_body(key_ref, bat_ref, nm_ref, sc_ref, nmout_ref, smul_ref,
                 rankT_ref, keptT_ref):
    key = key_ref[...]
    bat = bat_ref[...]
    nm = nm_ref[...]
    keyT = key.T
    batT = bat.T
    ii = lax.broadcasted_iota(jnp.int32, (128, 128), 0)
    jj = lax.broadcasted_iota(jnp.int32, (128, 128), 1)
    lt = jj < ii
    one = jnp.ones((128, 128), _f32)
    zero = jnp.zeros((128, 128), _f32)
    for b in range(_NB):
        kcol = keyT[:, b:b + 1]
        bcol = batT[:, b:b + 1]
        rank = jnp.zeros((128, 1), _f32)
        kept = jnp.zeros((128, 1), _f32)
        for bp in (b - 1, b, b + 1):
            if bp < 0 or bp >= _NB:
                continue
            krow = key[bp:bp + 1, :]
            brow = bat[bp:bp + 1, :]
            nrow = nm[bp:bp + 1, :]
            eq = bcol == brow
            if bp < b:
                cmp = krow >= kcol
            elif bp > b:
                cmp = krow > kcol
            else:
                cmp = (krow > kcol) | ((krow == kcol) & lt)
            rank += jnp.sum(jnp.where(eq & cmp, one, zero), axis=1,
                            keepdims=True)
            kept += jnp.sum(jnp.where(eq, jnp.broadcast_to(nrow, (128, 128)),
                                      zero), axis=1, keepdims=True)
        rankT_ref[:, b:b + 1] = rank
        keptT_ref[:, b:b + 1] = kept
    rank2 = rankT_ref[...].T
    kept2 = keptT_ref[...].T
    kf = jnp.ceil(_f32(0.8) * kept2)
    newmask = ((rank2 < kf) & (nm > 0.5)).astype(_f32)
    nmout_ref[...] = newmask
    smul_ref[...] = sc_ref[...] * newmask


def _select_call(key2, bat2, nm2, sc2):
    return pl.pallas_call(
        _select_body,
        out_shape=[
            jax.ShapeDtypeStruct((_NB, 128), _f32),
            jax.ShapeDtypeStruct((_NB, 128), _f32),
        ],
        scratch_shapes=[
            pltpu.VMEM((128, _NB), _f32),
            pltpu.VMEM((128, _NB), _f32),
        ],
    )(key2, bat2, nm2, sc2)


_G = 1024


def _gap_body(h1_ref, smul_ref, bat_ref, nm_ref, h2_ref, xsum_ref, cnt_ref):
    i = pl.program_id(0)
    h2 = h1_ref[...] * smul_ref[...][:, None]
    h2_ref[...] = h2
    bat = bat_ref[...]
    gi = lax.broadcasted_iota(jnp.int32, (B, _G), 0)
    M = (gi == bat[None, :]).astype(_f32)
    ps = jnp.dot(M, h2, preferred_element_type=_f32)
    pc = jnp.dot(M, nm_ref[...], preferred_element_type=_f32)

    @pl.when(i == 0)
    def _():
        xsum_ref[...] = ps
        cnt_ref[...] = pc

    @pl.when(i > 0)
    def _():
        xsum_ref[...] += ps
        cnt_ref[...] += pc


def _gap_call(h1, smul, bat, nm):
    grid = NPAD // _G
    return pl.pallas_call(
        _gap_body,
        grid=(grid,),
        in_specs=[
            pl.BlockSpec((_G, D), lambda i: (i, 0)),
            pl.BlockSpec((_G,), lambda i: (i,)),
            pl.BlockSpec((_G,), lambda i: (i,)),
            pl.BlockSpec((_G,), lambda i: (i,)),
        ],
        out_specs=[
            pl.BlockSpec((_G, D), lambda i: (i, 0)),
            pl.BlockSpec((B, D), lambda i: (0, 0)),
            pl.BlockSpec((B,), lambda i: (0,)),
        ],
        out_shape=[
            jax.ShapeDtypeStruct((NPAD, D), _f32),
            jax.ShapeDtypeStruct((B, D), _f32),
            jax.ShapeDtypeStruct((B,), _f32),
        ],
    )(h1, smul, bat, nm)


def _mlp_body(s1_ref, c1_ref, s2_ref, c2_ref, s3_ref, c3_ref,
              w1_ref, b1_ref, w2_ref, b2_ref, w3_ref, b3_ref, out_ref):
    z = (s1_ref[...] / jnp.maximum(c1_ref[...], 1.0)[:, None]
         + s2_ref[...] / jnp.maximum(c2_ref[...], 1.0)[:, None]
         + s3_ref[...] / jnp.maximum(c3_ref[...], 1.0)[:, None])
    z = jnp.maximum(jnp.dot(z, w1_ref[...], preferred_element_type=_f32)
                    + b1_ref[...][None, :], 0.0)
    z = jnp.maximum(jnp.dot(z, w2_ref[...], preferred_element_type=_f32)
                    + b2_ref[...][None, :], 0.0)
    z = jnp.dot(z, w3_ref[...], preferred_element_type=_f32) + b3_ref[...][None, :]
    out_ref[...] = jax.nn.sigmoid(z[:, 0])


def _mlp_call(s1, c1, s2, c2, s3, c3, W1, b1, W2, b2, W3, b3):
    return pl.pallas_call(
        _mlp_body,
        out_shape=jax.ShapeDtypeStruct((B,), _f32),
    )(s1, c1, s2, c2, s3, c3, W1, b1, W2, b2, W3, b3)


def kernel(x, edge_index, batch, emb, Wl1, bl1, Wr1, p1, Wl2, bl2, Wr2, p2,
           Wl3, bl3, Wr3, p3, W1, b1, W2, b2, W3, b3):
    npad = NPAD - N
    epad = EPAD - E
    xi = jnp.concatenate([x[:, 0],
                          jnp.arange(npad, dtype=jnp.int32)])
    pe = jnp.arange(epad, dtype=jnp.int32) % npad + N
    srcp = jnp.concatenate([edge_index[0], pe]).reshape(EPAD // 128, 128)
    dstp = jnp.concatenate([edge_index[1], pe]).reshape(EPAD // 128, 128)
    batp = jnp.concatenate([batch, jnp.full((npad,), 1000, jnp.int32)])
    bat2 = batp.reshape(_NB, 128)
    nm = jnp.concatenate([jnp.ones((N,), _f32), jnp.zeros((npad,), _f32)])

    h = _emb_call(emb, xi)

    sums = []
    for (Wl, bl, Wr, p) in ((Wl1, bl1, Wr1, p1), (Wl2, bl2, Wr2, p2),
                            (Wl3, bl3, Wr3, p3)):
        aggP, degP = _agg_call(srcp, dstp, h, nm)
        h1, score, key = _sage_call(aggP, degP, h, nm, Wl, bl, Wr, p)
        nm2, smul2 = _select_call(key.reshape(_NB, 128), bat2,
                                  nm.reshape(_NB, 128),
                                  score.reshape(_NB, 128))
        nm = nm2.reshape(NPAD)
        h, xsum, cnt = _gap_call(h1, smul2.reshape(NPAD), batp, nm)
        sums.append((xsum, cnt))

    (s1, c1), (s2, c2), (s3, c3) = sums
    return _mlp_call(s1, c1, s2, c2, s3, c3, W1, b1, W2, b2, W3, b3)


# speedup vs baseline: 1.2980x; 1.2980x over previous
import functools

import jax
import jax.numpy as jnp
from jax import lax
from jax.experimental import pallas as pl
from jax.experimental.pallas import tpu as pltpu
from jax.experimental.pallas import tpu_sc as plsc

N = 10000
NPAD = 10240
E = 320000
EPAD = 327680
B = 512
D = 128
VOCAB = 100000
NC = 2
NS = 16
NW = NC * NS

_f32 = jnp.float32


def _emb_body(emb_hbm, idx_hbm, out_hbm, idx_v, rows_v, sem):
    c = lax.axis_index("c")
    s = lax.axis_index("s")
    wid = s * NC + c
    pltpu.sync_copy(idx_hbm.at[pl.ds(wid * 320, 320)], idx_v)
    for g in range(5):
        pltpu.async_copy(emb_hbm.at[idx_v.at[pl.ds(g * 64, 64)]],
                         rows_v, sem).wait()
        pltpu.sync_copy(rows_v, out_hbm.at[pl.ds(wid * 320 + g * 64, 64)])


@functools.cache
def _emb_kernel():
    return pl.kernel(
        _emb_body,
        out_type=jax.ShapeDtypeStruct((NPAD, D), _f32),
        mesh=plsc.VectorSubcoreMesh(
            core_axis_name="c", subcore_axis_name="s",
            num_cores=NC, num_subcores=NS,
        ),
        scratch_types=[
            pltpu.VMEM((320,), jnp.int32),
            pltpu.VMEM((64, D), _f32),
            pltpu.SemaphoreType.DMA,
        ],
    )


def _emb_call(emb, xi2):
    return _emb_kernel()(emb, xi2)


def _agg_body(src_hbm, dst_hbm, h_hbm, agg_hbm,
              src_v, dst_v, rows_v, agg_sh, sem, sem2, ssem, ssem2):
    c = lax.axis_index("c")
    s = lax.axis_index("s")
    wid = s * NC + c
    zero16 = jnp.zeros((16,), _f32)

    def _zr(i, carry):
        rows_v[0, i // 8, pl.ds((i % 8) * 16, 16)] = zero16
        return carry
    lax.fori_loop(0, D * D // 16, _zr, 0)
    for z in range(5):
        pltpu.sync_copy(rows_v.at[0], agg_sh.at[pl.ds(s * 640 + z * 128, 128)])
    plsc.subcore_barrier()

    def _chunk(ch, carry):
        rbase = wid * 80 + ch * 40
        pltpu.sync_copy(src_hbm.at[pl.ds(rbase, 40)], src_v)
        pltpu.sync_copy(dst_hbm.at[pl.ds(rbase, 40)], dst_v)
        pltpu.async_copy(h_hbm.at[src_v.at[0]], rows_v.at[0], sem)

        def _pair(gg, carry2):
            g0 = gg * 2
            g1 = g0 + 1
            g2 = jnp.minimum(g0 + 2, 39)
            pltpu.async_copy(h_hbm.at[src_v.at[g1]], rows_v.at[1], sem2)
            pltpu.make_async_copy(h_hbm.at[src_v.at[g0]], rows_v.at[0],
                                  sem).wait()
            pltpu.sync_copy(rows_v.at[0], agg_sh.at[dst_v.at[g0]], add=True)
            pltpu.async_copy(h_hbm.at[src_v.at[g2]], rows_v.at[0], sem)
            pltpu.make_async_copy(h_hbm.at[src_v.at[g1]], rows_v.at[1],
                                  sem2).wait()
            pltpu.sync_copy(rows_v.at[1], agg_sh.at[dst_v.at[g1]], add=True)
            return carry2
        lax.fori_loop(0, 20, _pair, 0)
        pltpu.make_async_copy(h_hbm.at[src_v.at[0]], rows_v.at[0], sem).wait()
        return carry
    lax.fori_loop(0, 2, _chunk, 0)

    plsc.subcore_barrier()
    pltpu.sync_copy(agg_sh.at[pl.ds(s * 640, 640)],
                    agg_hbm.at[c, pl.ds(s * 640, 640)])


@functools.cache
def _agg_kernel():
    return pl.kernel(
        _agg_body,
        out_type=jax.ShapeDtypeStruct((NC, NPAD, D), _f32),
        mesh=plsc.VectorSubcoreMesh(
            core_axis_name="c", subcore_axis_name="s",
            num_cores=NC, num_subcores=NS,
        ),
        scratch_types=[
            pltpu.VMEM((40, 128), jnp.int32),
            pltpu.VMEM((40, 128), jnp.int32),
            pltpu.VMEM((2, 128, D), _f32),
            pltpu.VMEM_SHARED((NPAD, D), _f32),
            pltpu.SemaphoreType.DMA,
            pltpu.SemaphoreType.DMA,
            pltpu.SemaphoreType.DMA,
            pltpu.SemaphoreType.DMA,
        ],
        compiler_params=pltpu.CompilerParams(needs_layout_passes=False),
    )


def _deg_body(src_hbm, dst_hbm, nm_hbm, deg_hbm, src_v, dst_v, nm_v, deg_v):
    c = lax.axis_index("c")
    s = lax.axis_index("s")
    wid = s * NC + c
    zero16 = jnp.zeros((16,), _f32)

    def _zd(i, carry):
        deg_v[pl.ds(i * 16, 16)] = zero16
        return carry
    lax.fori_loop(0, NPAD // 16, _zd, 0)

    pltpu.sync_copy(nm_hbm, nm_v)
    pltpu.sync_copy(src_hbm.at[pl.ds(wid * 80, 80)], src_v)
    pltpu.sync_copy(dst_hbm.at[pl.ds(wid * 80, 80)], dst_v)

    def _deg(j, carry):
        si = src_v[j // 8, pl.ds((j % 8) * 16, 16)]
        vals = plsc.load_gather(nm_v, [si])
        di = dst_v[j // 8, pl.ds((j % 8) * 16, 16)]
        plsc.addupdate_scatter(deg_v, [di], vals)
        return carry
    lax.fori_loop(0, 640, _deg, 0)
    pltpu.sync_copy(deg_v, deg_hbm.at[pl.ds(wid * NPAD, NPAD)])


@functools.cache
def _deg_kernel():
    return pl.kernel(
        _deg_body,
        out_type=jax.ShapeDtypeStruct((NW * NPAD,), _f32),
        mesh=plsc.VectorSubcoreMesh(
            core_axis_name="c", subcore_axis_name="s",
            num_cores=NC, num_subcores=NS,
        ),
        scratch_types=[
            pltpu.VMEM((80, 128), jnp.int32),
            pltpu.VMEM((80, 128), jnp.int32),
            pltpu.VMEM((NPAD,), _f32),
            pltpu.VMEM((NPAD,), _f32),
        ],
        compiler_params=pltpu.CompilerParams(needs_layout_passes=False),
    )


def _agg_call(srcp, dstp, h, nm):
    aggP = _agg_kernel()(srcp, dstp, h)
    degP = _deg_kernel()(srcp, dstp, nm)
    return aggP, degP.reshape(NW, NPAD)


_NB = NPAD // 128


def _layer_body(aggP_ref, degP_ref, h_ref, nm2_ref, bat2_ref, batf_ref,
                wl_ref, bl_ref, wr_ref, p_ref,
                h2_ref, nmout_ref, xsum_ref, cnt_ref, rankT_ref, keptT_ref,
                colA_ref, colB_ref):
    agg = aggP_ref[0] + aggP_ref[1]
    deg = jnp.sum(degP_ref[...], axis=0)
    rcp = 1.0 / jnp.maximum(deg, 1.0)[:, None]
    nm2 = nm2_ref[...]
    h = h_ref[...]
    mean = agg * rcp
    out = (jnp.dot(mean, wl_ref[...], preferred_element_type=_f32)
           + bl_ref[...][None, :]
           + jnp.dot(h, wr_ref[...], preferred_element_type=_f32))
    colA_ref[...] = nm2.reshape(NPAD)
    h1 = jnp.maximum(out, 0.0) * colA_ref[...][:, None]
    p = p_ref[...]
    pn = jax.lax.rsqrt(jnp.sum(p * p))
    sc2 = jnp.tanh(jnp.dot(h1, p, preferred_element_type=_f32)
                   * pn).reshape(_NB, 128)
    key = jnp.where(nm2 > 0.5, sc2, -2.0)
    bat = bat2_ref[...]
    keyT = key.T
    batT = bat.T
    ii = lax.broadcasted_iota(jnp.int32, (128, 128), 0)
    jj = lax.broadcasted_iota(jnp.int32, (128, 128), 1)
    lt = jj < ii
    one = jnp.ones((128, 128), _f32)
    zero = jnp.zeros((128, 128), _f32)
    for b in range(_NB):
        kcol = keyT[:, b:b + 1]
        bcol = batT[:, b:b + 1]
        rank = jnp.zeros((128, 1), _f32)
        kept = jnp.zeros((128, 1), _f32)
        for bp in (b - 1, b, b + 1):
            if bp < 0 or bp >= _NB:
                continue
            krow = key[bp:bp + 1, :]
            brow = bat[bp:bp + 1, :]
            nrow = nm2[bp:bp + 1, :]
            eq = bcol == brow
            if bp < b:
                cmp = krow >= kcol
            elif bp > b:
                cmp = krow > kcol
            else:
                cmp = (krow > kcol) | ((krow == kcol) & lt)
            rank += jnp.sum(jnp.where(eq & cmp, one, zero), axis=1,
                            keepdims=True)
            kept += jnp.sum(jnp.where(eq, jnp.broadcast_to(nrow, (128, 128)),
                                      zero), axis=1, keepdims=True)
        rankT_ref[:, b:b + 1] = rank
        keptT_ref[:, b:b + 1] = kept
    rank2 = rankT_ref[...].T
    kept2 = keptT_ref[...].T
    kf = jnp.ceil(_f32(0.8) * kept2)
    newmask = ((rank2 < kf) & (nm2 > 0.5)).astype(_f32)
    nmout_ref[...] = newmask
    colB_ref[...] = (sc2 * newmask).reshape(NPAD)
    h2 = h1 * colB_ref[...][:, None]
    h2_ref[...] = h2
    batf = batf_ref[...]
    colA_ref[...] = newmask.reshape(NPAD)
    nmf = colA_ref[...]
    xs = jnp.zeros((B, D), _f32)
    ct = jnp.zeros((B,), _f32)
    for i in range(NPAD // 1024):
        batb = batf[i * 1024:(i + 1) * 1024]
        M = (lax.broadcasted_iota(jnp.int32, (B, 1024), 0)
             == batb[None, :]).astype(_f32)
        xs = xs + jnp.dot(M, h2[i * 1024:(i + 1) * 1024],
                          preferred_element_type=_f32)
        ct = ct + jnp.dot(M, nmf[i * 1024:(i + 1) * 1024],
                          preferred_element_type=_f32)
    xsum_ref[...] = xs
    cnt_ref[...] = ct


def _layer_call(aggP, degP, h, nm2, bat2, batf, Wl, bl, Wr, p):
    return pl.pallas_call(
        _layer_body,
        out_shape=[
            jax.ShapeDtypeStruct((NPAD, D), _f32),
            jax.ShapeDtypeStruct((_NB, 128), _f32),
            jax.ShapeDtypeStruct((B, D), _f32),
            jax.ShapeDtypeStruct((B,), _f32),
        ],
        scratch_shapes=[
            pltpu.VMEM((128, _NB), _f32),
            pltpu.VMEM((128, _NB), _f32),
            pltpu.VMEM((NPAD,), _f32),
            pltpu.VMEM((NPAD,), _f32),
        ],
    )(aggP, degP, h, nm2, bat2, batf, Wl, bl, Wr, p)


def _mlp_body(s1_ref, c1_ref, s2_ref, c2_ref, s3_ref, c3_ref,
              w1_ref, b1_ref, w2_ref, b2_ref, w3_ref, b3_ref, out_ref):
    z = (s1_ref[...] / jnp.maximum(c1_ref[...], 1.0)[:, None]
         + s2_ref[...] / jnp.maximum(c2_ref[...], 1.0)[:, None]
         + s3_ref[...] / jnp.maximum(c3_ref[...], 1.0)[:, None])
    z = jnp.maximum(jnp.dot(z, w1_ref[...], preferred_element_type=_f32)
                    + b1_ref[...][None, :], 0.0)
    z = jnp.maximum(jnp.dot(z, w2_ref[...], preferred_element_type=_f32)
                    + b2_ref[...][None, :], 0.0)
    z = jnp.dot(z, w3_ref[...], preferred_element_type=_f32) + b3_ref[...][None, :]
    out_ref[...] = jax.nn.sigmoid(z[:, 0])


def _mlp_call(s1, c1, s2, c2, s3, c3, W1, b1, W2, b2, W3, b3):
    return pl.pallas_call(
        _mlp_body,
        out_shape=jax.ShapeDtypeStruct((B,), _f32),
    )(s1, c1, s2, c2, s3, c3, W1, b1, W2, b2, W3, b3)


def kernel(x, edge_index, batch, emb, Wl1, bl1, Wr1, p1, Wl2, bl2, Wr2, p2,
           Wl3, bl3, Wr3, p3, W1, b1, W2, b2, W3, b3):
    npad = NPAD - N
    epad = EPAD - E
    xi = jnp.concatenate([x[:, 0],
                          jnp.arange(npad, dtype=jnp.int32)])
    pe = jnp.arange(epad, dtype=jnp.int32) % npad + N
    srcp = jnp.concatenate([edge_index[0], pe]).reshape(EPAD // 128, 128)
    dstp = jnp.concatenate([edge_index[1], pe]).reshape(EPAD // 128, 128)
    batp = jnp.concatenate([batch, jnp.full((npad,), 1000, jnp.int32)])
    bat2 = batp.reshape(_NB, 128)
    nm = jnp.concatenate([jnp.ones((N,), _f32), jnp.zeros((npad,), _f32)])

    h = _emb_call(emb, xi)

    sums = []
    nm2 = nm.reshape(_NB, 128)
    for (Wl, bl, Wr, p) in ((Wl1, bl1, Wr1, p1), (Wl2, bl2, Wr2, p2),
                            (Wl3, bl3, Wr3, p3)):
        aggP, degP = _agg_call(srcp, dstp, h, nm2.reshape(NPAD))
        h, nm2, xsum, cnt = _layer_call(aggP, degP, h, nm2, bat2, batp,
                                        Wl, bl, Wr, p)
        sums.append((xsum, cnt))

    (s1, c1), (s2, c2), (s3, c3) = sums
    return _mlp_call(s1, c1, s2, c2, s3, c3, W1, b1, W2, b2, W3, b3)
